# baseline (speedup 1.0000x reference)
import functools

import jax
import jax.numpy as jnp
from jax.experimental import pallas as pl
from jax.experimental.pallas import tpu as pltpu
from jax.experimental.pallas import tpu_sc as plsc

LAYERS = 6
HEADS = 8
LH = LAYERS * HEADS
L = 2048
T = 512

SC_HEADS = 8
TC_HEADS = LH - SC_HEADS
NTILES = 32
ROWS_PER_TILE = L // NTILES
LANES = 16

SCAN_BLOCK = 4


def _scan_kernel(x_ref, score_ref):
    rmax = jnp.max(x_ref[...], axis=-1, keepdims=True)
    score_ref[...] = jnp.mean(rmax, axis=1, keepdims=True)


def _lane_shuffle(v, idx):
    return jax.lax.gather(
        v, idx[:, None],
        jax.lax.GatherDimensionNumbers(
            offset_dims=(), collapsed_slice_dims=(0,), start_index_map=(0,)),
        slice_sizes=(1,),
        mode=jax.lax.GatherScatterMode.PROMISE_IN_BOUNDS)


def _sc_scan_body(a_hbm, out_hbm, buf, outvec):
    c = jax.lax.axis_index("c")
    s = jax.lax.axis_index("s")
    wid = s * 2 + c
    base = wid * ROWS_PER_TILE
    res = jnp.zeros((LANES,), jnp.float32)
    lane = jax.lax.iota(jnp.int32, LANES)
    for h in range(SC_HEADS):
        pltpu.sync_copy(
            a_hbm.at[TC_HEADS + h, pl.ds(base, ROWS_PER_TILE), :], buf)

        def row_body(r, acc, _buf=buf, _lane=lane):
            m = _buf[r, pl.ds(0, LANES)]
            for i in range(1, T // LANES):
                m = jnp.maximum(m, _buf[r, pl.ds(i * LANES, LANES)])
            for k in (8, 4, 2, 1):
                m = jnp.maximum(m, _lane_shuffle(m, _lane ^ k))
            return acc + m

        acc = jax.lax.fori_loop(0, ROWS_PER_TILE, row_body,
                                jnp.zeros((LANES,), jnp.float32))
        res = jnp.where(lane == h, acc, res)
    outvec[...] = res
    pltpu.sync_copy(outvec, out_hbm.at[wid])


def _finalize_kernel(widx_ref, x_ref, score_ref, part_ref, dur_ref, focus_ref):
    del widx_ref
    x = x_ref[0]
    rmax = jnp.max(x, axis=-1, keepdims=True)
    iota_t = jax.lax.broadcasted_iota(jnp.int32, (L, T), 1)
    ridx = jnp.min(jnp.where(x == rmax, iota_t, T), axis=-1, keepdims=True)
    eq = (ridx == iota_t).astype(jnp.int32)
    dur_ref[0, :] = jnp.sum(eq, axis=0)
    scores_tc = score_ref[:, :, 0]
    scores_sc = jnp.sum(part_ref[:, :SC_HEADS], axis=0, keepdims=True) / L
    m_tc = jnp.max(scores_tc, axis=(0, 1), keepdims=True)
    m_sc = jnp.max(scores_sc, axis=(0, 1), keepdims=True)
    focus_ref[:, :] = jnp.maximum(m_tc, m_sc)


_sc_scan = functools.partial(
    pl.kernel,
    out_type=jax.ShapeDtypeStruct((NTILES, LANES), jnp.float32),
    mesh=plsc.VectorSubcoreMesh(core_axis_name="c", subcore_axis_name="s",
                                num_cores=2, num_subcores=16),
    scratch_types=[
        pltpu.VMEM((ROWS_PER_TILE, T), jnp.float32),
        pltpu.VMEM((LANES,), jnp.float32),
    ],
    compiler_params=pltpu.CompilerParams(use_tc_tiling_on_sc=True),
)(_sc_scan_body)


def kernel(att_ws):
    a = att_ws.reshape(LH, L, T)
    partials = _sc_scan(a)
    scores_tc = pl.pallas_call(
        _scan_kernel,
        grid=(TC_HEADS // SCAN_BLOCK,),
        in_specs=[pl.BlockSpec((SCAN_BLOCK, L, T), lambda i: (i, 0, 0))],
        out_specs=pl.BlockSpec((SCAN_BLOCK, 1, 1), lambda i: (i, 0, 0)),
        out_shape=jax.ShapeDtypeStruct((TC_HEADS, 1, 1), jnp.float32),
    )(a)
    scores_all = jnp.concatenate(
        [scores_tc.reshape(TC_HEADS),
         jnp.sum(partials[:, :SC_HEADS], axis=0) / L])
    widx = jnp.argmax(scores_all).astype(jnp.int32).reshape(1)
    durations, focus = pl.pallas_call(
        _finalize_kernel,
        grid_spec=pltpu.PrefetchScalarGridSpec(
            num_scalar_prefetch=1,
            grid=(1,),
            in_specs=[
                pl.BlockSpec((1, L, T), lambda i, w: (w[0], 0, 0)),
                pl.BlockSpec((TC_HEADS, 1, 1), lambda i, w: (0, 0, 0)),
                pl.BlockSpec((NTILES, LANES), lambda i, w: (0, 0)),
            ],
            out_specs=[
                pl.BlockSpec((1, T), lambda i, w: (0, 0)),
                pl.BlockSpec((1, 1), lambda i, w: (0, 0)),
            ],
        ),
        out_shape=[
            jax.ShapeDtypeStruct((1, T), jnp.int32),
            jax.ShapeDtypeStruct((1, 1), jnp.float32),
        ],
    )(widx, a, scores_tc, partials)
    return durations.reshape(T), focus.reshape(())


# speedup vs baseline: 1.2300x; 1.2300x over previous
import jax
import jax.numpy as jnp
from jax.experimental import pallas as pl
from jax.experimental.pallas import tpu as pltpu

LAYERS = 6
HEADS = 8
LH = LAYERS * HEADS
L = 2048
T = 512


SCAN_BLOCK = 6


def _scan_kernel(x_ref, score_ref):
    rmax = jnp.max(x_ref[...], axis=-1, keepdims=True)
    score_ref[...] = jnp.mean(rmax, axis=1, keepdims=True)


def _finalize_kernel(widx_ref, x_ref, score_ref, dur_ref, focus_ref):
    del widx_ref
    x = x_ref[0]
    rmax = jnp.max(x, axis=-1, keepdims=True)
    iota_t = jax.lax.broadcasted_iota(jnp.int32, (L, T), 1)
    ridx = jnp.min(jnp.where(x == rmax, iota_t, T), axis=-1, keepdims=True)
    eq = (ridx == iota_t).astype(jnp.int32)
    dur_ref[0, :] = jnp.sum(eq, axis=0)
    scores = score_ref[:, :, 0]
    focus_ref[:, :] = jnp.max(scores, axis=(0, 1), keepdims=True)


def kernel(att_ws):
    a = att_ws.reshape(LH, L, T)
    scores = pl.pallas_call(
        _scan_kernel,
        grid=(LH // SCAN_BLOCK,),
        in_specs=[pl.BlockSpec((SCAN_BLOCK, L, T), lambda i: (i, 0, 0))],
        out_specs=pl.BlockSpec((SCAN_BLOCK, 1, 1), lambda i: (i, 0, 0)),
        out_shape=jax.ShapeDtypeStruct((LH, 1, 1), jnp.float32),
    )(a)
    widx = jnp.argmax(scores.reshape(LH)).astype(jnp.int32).reshape(1)
    durations, focus = pl.pallas_call(
        _finalize_kernel,
        grid_spec=pltpu.PrefetchScalarGridSpec(
            num_scalar_prefetch=1,
            grid=(1,),
            in_specs=[
                pl.BlockSpec((1, L, T), lambda i, w: (w[0], 0, 0)),
                pl.BlockSpec((LH, 1, 1), lambda i, w: (0, 0, 0)),
            ],
            out_specs=[
                pl.BlockSpec((1, T), lambda i, w: (0, 0)),
                pl.BlockSpec((1, 1), lambda i, w: (0, 0)),
            ],
        ),
        out_shape=[
            jax.ShapeDtypeStruct((1, T), jnp.int32),
            jax.ShapeDtypeStruct((1, 1), jnp.float32),
        ],
    )(widx, a, scores)
    return durations.reshape(T), focus.reshape(())
